# baseline (speedup 1.0000x reference)
import functools

import jax
import jax.numpy as jnp
from jax.experimental import pallas as pl
from jax.experimental.pallas import tpu as pltpu

NUM_EXPERTS = 8
TOP_K = 2
D_MODEL = 1024
D_FF = 2048

TM = 512


def _group_body(sched_ref, x_ref, w1_ref, w3_ref, w2_ref, y_ref):
    g = pl.program_id(0)
    lo = sched_ref[2, g]
    hi = sched_ref[3, g]
    first = sched_ref[4, g]
    t = sched_ref[0, g]

    @pl.when(hi > lo)
    def _compute():
        xb = x_ref[...]
        h1 = jnp.dot(xb, w1_ref[0], preferred_element_type=jnp.float32)
        h3 = jnp.dot(xb, w3_ref[0], preferred_element_type=jnp.float32)
        h = (h1 * jax.nn.sigmoid(h1)) * h3
        y = jnp.dot(h, w2_ref[0], preferred_element_type=jnp.float32)
        rows = t * TM + jax.lax.broadcasted_iota(jnp.int32, (TM, 1), 0)
        mask = (rows >= lo) & (rows < hi)
        contrib = jnp.where(mask, y, 0.0)

        @pl.when(first == 1)
        def _init():
            y_ref[...] = contrib

        @pl.when(first == 0)
        def _acc():
            y_ref[...] += contrib


def kernel(x, Wg, w1, w3, w2):
    B, S, D = x.shape
    T = B * S
    A = T * TOP_K
    nt = A // TM
    G = nt + NUM_EXPERTS - 1

    gate_logits = jnp.einsum('bsd,de->bse', x, Wg)
    weights, selected = jax.lax.top_k(gate_logits, TOP_K)
    weights = jax.nn.softmax(weights, axis=2)

    e_flat = selected.reshape(A).astype(jnp.int32)
    g_flat = weights.reshape(A)

    CH = 128
    nch = A // CH
    onehot = (jnp.arange(NUM_EXPERTS, dtype=jnp.int32)[:, None]
              == e_flat[None, :]).astype(jnp.float32)
    ohc = onehot.reshape(NUM_EXPERTS, nch, CH)
    ii = jnp.arange(CH)
    u_incl = (ii[:, None] <= ii[None, :]).astype(jnp.float32)
    within = jnp.einsum('enc,cd->end', ohc, u_incl)
    chunk_tot = within[..., -1]
    jj = jnp.arange(nch)
    u_excl = (jj[:, None] < jj[None, :]).astype(jnp.float32)
    chunk_base = jnp.einsum('en,nm->em', chunk_tot, u_excl)
    sizes_f = chunk_tot.sum(-1)
    starts_f = jnp.concatenate([jnp.zeros((1,), jnp.float32),
                                jnp.cumsum(sizes_f)[:-1]])
    posmap = within + chunk_base[:, :, None] - 1.0 \
        + starts_f[:, None, None]
    pos = jnp.sum(posmap.reshape(NUM_EXPERTS, A) * onehot,
                  axis=0).astype(jnp.int32)
    inv = pos
    sizes = sizes_f.astype(jnp.int32)
    ends = jnp.cumsum(sizes)
    starts = ends - sizes


    t_start = starts // TM
    t_last = jnp.maximum(ends - 1, 0) // TM
    touched = jnp.where(sizes > 0, t_last - t_start + 1, 0)
    pair_end = jnp.cumsum(touched)
    pair_start = pair_end - touched

    gidx = jnp.arange(G, dtype=jnp.int32)
    e_of_g = jnp.searchsorted(pair_end, gidx, side='right').astype(jnp.int32)
    valid = e_of_g < NUM_EXPERTS
    e_cl = jnp.minimum(e_of_g, NUM_EXPERTS - 1)
    last_e = jnp.searchsorted(pair_end, pair_end[-1] - 1,
                              side='right').astype(jnp.int32)
    e_g = jnp.where(valid, e_cl, last_e)
    t_g = jnp.where(valid, t_start[e_cl] + (gidx - pair_start[e_cl]), nt - 1)
    lo_g = jnp.where(valid, jnp.maximum(starts[e_g], t_g * TM), 0)
    hi_g = jnp.where(valid, jnp.minimum(ends[e_g], (t_g + 1) * TM), 0)
    first_g = jnp.concatenate([
        jnp.ones((1,), jnp.int32),
        (t_g[1:] != t_g[:-1]).astype(jnp.int32),
    ])
    sched = jnp.stack([t_g, e_g, lo_g, hi_g, first_g])

    xf = x.reshape(T, D)
    xf2 = jnp.broadcast_to(xf[:, None, :], (T, TOP_K, D)).reshape(A, D)
    x_sorted = jnp.zeros((A, D), jnp.float32).at[pos].set(xf2)

    grid_spec = pltpu.PrefetchScalarGridSpec(
        num_scalar_prefetch=1,
        grid=(G,),
        in_specs=[
            pl.BlockSpec((TM, D_MODEL), lambda g, s: (s[0, g], 0)),
            pl.BlockSpec((1, D_MODEL, D_FF), lambda g, s: (s[1, g], 0, 0)),
            pl.BlockSpec((1, D_MODEL, D_FF), lambda g, s: (s[1, g], 0, 0)),
            pl.BlockSpec((1, D_FF, D_MODEL), lambda g, s: (s[1, g], 0, 0)),
        ],
        out_specs=pl.BlockSpec((TM, D_MODEL), lambda g, s: (s[0, g], 0)),
    )
    y_sorted = pl.pallas_call(
        _group_body,
        grid_spec=grid_spec,
        out_shape=jax.ShapeDtypeStruct((A, D_MODEL), jnp.float32),
        compiler_params=pltpu.CompilerParams(
            vmem_limit_bytes=100 * 1024 * 1024),
    )(sched, x_sorted, w1, w3, w2)

    inv2 = inv.reshape(T, TOP_K)
    wt = weights.reshape(T, TOP_K)
    out = wt[:, 0:1] * jnp.take(y_sorted, inv2[:, 0], axis=0) \
        + wt[:, 1:2] * jnp.take(y_sorted, inv2[:, 1], axis=0)
    return out.reshape(B, S, D)


# speedup vs baseline: 1.7032x; 1.7032x over previous
import functools

import jax
import jax.numpy as jnp
from jax import lax
from jax.experimental import pallas as pl
from jax.experimental.pallas import tpu as pltpu
from jax.experimental.pallas import tpu_sc as plsc

NUM_EXPERTS = 8
TOP_K = 2
D_MODEL = 1024
D_FF = 2048

TM = 512
LANES = 16
NW = 32
REORDER_CT = 64
COMBINE_CT = 32


def _group_body(sched_ref, x_ref, w1_ref, w3_ref, w2_ref, y_ref):
    g = pl.program_id(0)
    lo = sched_ref[2, g]
    hi = sched_ref[3, g]
    first = sched_ref[4, g]
    t = sched_ref[0, g]

    @pl.when(hi > lo)
    def _compute():
        xb = x_ref[...]
        h1 = jnp.dot(xb, w1_ref[0], preferred_element_type=jnp.float32)
        h3 = jnp.dot(xb, w3_ref[0], preferred_element_type=jnp.float32)
        h = (h1 * jax.nn.sigmoid(h1)) * h3
        y = jnp.dot(h, w2_ref[0], preferred_element_type=jnp.float32)
        rows = t * TM + jax.lax.broadcasted_iota(jnp.int32, (TM, 1), 0)
        mask = (rows >= lo) & (rows < hi)
        contrib = jnp.where(mask, y, 0.0)

        @pl.when(first == 1)
        def _init():
            y_ref[...] = contrib

        @pl.when(first == 0)
        def _acc():
            y_ref[...] += contrib


_SC_MESH = plsc.VectorSubcoreMesh(core_axis_name="c", subcore_axis_name="s")


def _sc_wid():
    return lax.axis_index("s") * 2 + lax.axis_index("c")


def _reorder_body(xf, pos2, xs, xrow_v, idxe_v, idxo_v, sem):
    T = xf.shape[0]
    per_w = T // NW
    wid = _sc_wid()

    def chunk(i, _):
        base = wid * per_w + i * REORDER_CT
        pltpu.sync_copy(pos2.at[0, pl.ds(base, REORDER_CT)], idxe_v)
        pltpu.sync_copy(pos2.at[1, pl.ds(base, REORDER_CT)], idxo_v)
        pltpu.sync_copy(xf.at[pl.ds(base, REORDER_CT)], xrow_v)
        c1 = pltpu.async_copy(xrow_v, xs.at[idxe_v], sem)
        c2 = pltpu.async_copy(xrow_v, xs.at[idxo_v], sem)
        c1.wait()
        c2.wait()
        return 0

    lax.fori_loop(0, per_w // REORDER_CT, chunk, 0)


def _combine_body(ys, pos2, wtb, out, idxe_v, idxo_v, wte_v, wto_v,
                  rows0_v, rows1_v, sem):
    T = out.shape[0]
    per_w = T // NW
    wid = _sc_wid()

    def chunk(i, _):
        base = wid * per_w + i * COMBINE_CT
        pltpu.sync_copy(pos2.at[0, pl.ds(base, COMBINE_CT)], idxe_v)
        pltpu.sync_copy(pos2.at[1, pl.ds(base, COMBINE_CT)], idxo_v)
        pltpu.sync_copy(wtb.at[0, pl.ds(base, COMBINE_CT)], wte_v)
        pltpu.sync_copy(wtb.at[1, pl.ds(base, COMBINE_CT)], wto_v)
        c1 = pltpu.async_copy(ys.at[idxe_v], rows0_v, sem)
        c2 = pltpu.async_copy(ys.at[idxo_v], rows1_v, sem)
        c1.wait()
        c2.wait()

        def row(r, _):
            w0 = wte_v[r, :]
            w1 = wto_v[r, :]
            for k in range(D_MODEL // LANES):
                sl = pl.ds(k * LANES, LANES)
                rows0_v[r, sl] = w0 * rows0_v[r, sl] + w1 * rows1_v[r, sl]
            return 0

        lax.fori_loop(0, COMBINE_CT, row, 0)
        pltpu.sync_copy(rows0_v, out.at[pl.ds(base, COMBINE_CT)])
        return 0

    lax.fori_loop(0, per_w // COMBINE_CT, chunk, 0)


def kernel(x, Wg, w1, w3, w2):
    B, S, D = x.shape
    T = B * S
    A = T * TOP_K
    nt = A // TM
    G = nt + NUM_EXPERTS - 1

    gate_logits = jnp.einsum('bsd,de->bse', x, Wg)
    weights, selected = jax.lax.top_k(gate_logits, TOP_K)
    weights = jax.nn.softmax(weights, axis=2)

    e_flat = selected.reshape(A).astype(jnp.int32)

    CH = 128
    nch = A // CH
    onehot = (jnp.arange(NUM_EXPERTS, dtype=jnp.int32)[:, None]
              == e_flat[None, :]).astype(jnp.float32)
    ohc = onehot.reshape(NUM_EXPERTS, nch, CH)
    ii = jnp.arange(CH)
    u_incl = (ii[:, None] <= ii[None, :]).astype(jnp.float32)
    within = jnp.einsum('enc,cd->end', ohc, u_incl)
    chunk_tot = within[..., -1]
    jj = jnp.arange(nch)
    u_excl = (jj[:, None] < jj[None, :]).astype(jnp.float32)
    chunk_base = jnp.einsum('en,nm->em', chunk_tot, u_excl)
    sizes_f = chunk_tot.sum(-1)
    starts_f = jnp.concatenate([jnp.zeros((1,), jnp.float32),
                                jnp.cumsum(sizes_f)[:-1]])
    posmap = within + chunk_base[:, :, None] - 1.0 \
        + starts_f[:, None, None]
    pos = jnp.sum(posmap.reshape(NUM_EXPERTS, A) * onehot,
                  axis=0).astype(jnp.int32)
    pos2 = pos.reshape(T, TOP_K).T
    sizes = sizes_f.astype(jnp.int32)
    ends = jnp.cumsum(sizes)
    starts = ends - sizes

    t_start = starts // TM
    t_last = jnp.maximum(ends - 1, 0) // TM
    touched = jnp.where(sizes > 0, t_last - t_start + 1, 0)
    pair_end = jnp.cumsum(touched)
    pair_start = pair_end - touched

    gidx = jnp.arange(G, dtype=jnp.int32)
    e_of_g = jnp.searchsorted(pair_end, gidx, side='right').astype(jnp.int32)
    valid = e_of_g < NUM_EXPERTS
    e_cl = jnp.minimum(e_of_g, NUM_EXPERTS - 1)
    last_e = jnp.searchsorted(pair_end, pair_end[-1] - 1,
                              side='right').astype(jnp.int32)
    e_g = jnp.where(valid, e_cl, last_e)
    t_g = jnp.where(valid, t_start[e_cl] + (gidx - pair_start[e_cl]), nt - 1)
    lo_g = jnp.where(valid, jnp.maximum(starts[e_g], t_g * TM), 0)
    hi_g = jnp.where(valid, jnp.minimum(ends[e_g], (t_g + 1) * TM), 0)
    first_g = jnp.concatenate([
        jnp.ones((1,), jnp.int32),
        (t_g[1:] != t_g[:-1]).astype(jnp.int32),
    ])
    sched = jnp.stack([t_g, e_g, lo_g, hi_g, first_g])

    xf = x.reshape(T, D)
    wtb = jnp.broadcast_to(
        weights.reshape(T, TOP_K).T[:, :, None], (TOP_K, T, LANES))

    reorder = functools.partial(
        pl.kernel,
        out_type=jax.ShapeDtypeStruct((A, D_MODEL), jnp.float32),
        mesh=_SC_MESH,
        scratch_types=[
            pltpu.VMEM((REORDER_CT, D_MODEL), jnp.float32),
            pltpu.VMEM((REORDER_CT,), jnp.int32),
            pltpu.VMEM((REORDER_CT,), jnp.int32),
            pltpu.SemaphoreType.DMA,
        ],
    )(_reorder_body)
    x_sorted = reorder(xf, pos2)

    grid_spec = pltpu.PrefetchScalarGridSpec(
        num_scalar_prefetch=1,
        grid=(G,),
        in_specs=[
            pl.BlockSpec((TM, D_MODEL), lambda g, s: (s[0, g], 0)),
            pl.BlockSpec((1, D_MODEL, D_FF), lambda g, s: (s[1, g], 0, 0)),
            pl.BlockSpec((1, D_MODEL, D_FF), lambda g, s: (s[1, g], 0, 0)),
            pl.BlockSpec((1, D_FF, D_MODEL), lambda g, s: (s[1, g], 0, 0)),
        ],
        out_specs=pl.BlockSpec((TM, D_MODEL), lambda g, s: (s[0, g], 0)),
    )
    y_sorted = pl.pallas_call(
        _group_body,
        grid_spec=grid_spec,
        out_shape=jax.ShapeDtypeStruct((A, D_MODEL), jnp.float32),
        compiler_params=pltpu.CompilerParams(
            vmem_limit_bytes=100 * 1024 * 1024),
    )(sched, x_sorted, w1, w3, w2)

    combine = functools.partial(
        pl.kernel,
        out_type=jax.ShapeDtypeStruct((T, D_MODEL), jnp.float32),
        mesh=_SC_MESH,
        scratch_types=[
            pltpu.VMEM((COMBINE_CT,), jnp.int32),
            pltpu.VMEM((COMBINE_CT,), jnp.int32),
            pltpu.VMEM((COMBINE_CT, LANES), jnp.float32),
            pltpu.VMEM((COMBINE_CT, LANES), jnp.float32),
            pltpu.VMEM((COMBINE_CT, D_MODEL), jnp.float32),
            pltpu.VMEM((COMBINE_CT, D_MODEL), jnp.float32),
            pltpu.SemaphoreType.DMA,
        ],
    )(_combine_body)
    out = combine(y_sorted, pos2, wtb)

    return out.reshape(B, S, D)


# speedup vs baseline: 1.7079x; 1.0027x over previous
import functools

import jax
import jax.numpy as jnp
from jax import lax
from jax.experimental import pallas as pl
from jax.experimental.pallas import tpu as pltpu
from jax.experimental.pallas import tpu_sc as plsc

NUM_EXPERTS = 8
TOP_K = 2
D_MODEL = 1024
D_FF = 2048

TM = 512
LANES = 16
NW = 32
REORDER_CT = 64
COMBINE_CT = 32


def _group_body(sched_ref, x_ref, w1_ref, w3_ref, w2_ref, y_ref):
    g = pl.program_id(0)
    lo = sched_ref[2, g]
    hi = sched_ref[3, g]
    first = sched_ref[4, g]
    t = sched_ref[0, g]

    @pl.when(hi > lo)
    def _compute():
        xb = x_ref[...].astype(jnp.bfloat16)
        h1 = jnp.dot(xb, w1_ref[0].astype(jnp.bfloat16),
                     preferred_element_type=jnp.float32)
        h3 = jnp.dot(xb, w3_ref[0].astype(jnp.bfloat16),
                     preferred_element_type=jnp.float32)
        h = ((h1 * jax.nn.sigmoid(h1)) * h3).astype(jnp.bfloat16)
        y = jnp.dot(h, w2_ref[0].astype(jnp.bfloat16),
                    preferred_element_type=jnp.float32)
        rows = t * TM + jax.lax.broadcasted_iota(jnp.int32, (TM, 1), 0)
        mask = (rows >= lo) & (rows < hi)
        contrib = jnp.where(mask, y, 0.0)

        @pl.when(first == 1)
        def _init():
            y_ref[...] = contrib

        @pl.when(first == 0)
        def _acc():
            y_ref[...] += contrib


_SC_MESH = plsc.VectorSubcoreMesh(core_axis_name="c", subcore_axis_name="s")


def _sc_wid():
    return lax.axis_index("s") * 2 + lax.axis_index("c")


def _reorder_body(xf, pos2, xs, xrow_v, idxe_v, idxo_v, sem):
    T = xf.shape[0]
    per_w = T // NW
    wid = _sc_wid()

    def chunk(i, _):
        base = wid * per_w + i * REORDER_CT
        pltpu.sync_copy(pos2.at[0, pl.ds(base, REORDER_CT)], idxe_v)
        pltpu.sync_copy(pos2.at[1, pl.ds(base, REORDER_CT)], idxo_v)
        pltpu.sync_copy(xf.at[pl.ds(base, REORDER_CT)], xrow_v)
        c1 = pltpu.async_copy(xrow_v, xs.at[idxe_v], sem)
        c2 = pltpu.async_copy(xrow_v, xs.at[idxo_v], sem)
        c1.wait()
        c2.wait()
        return 0

    lax.fori_loop(0, per_w // REORDER_CT, chunk, 0)


def _combine_body(ys, pos2, wtb, out, idxe_v, idxo_v, wte_v, wto_v,
                  rows0_v, rows1_v, sem):
    T = out.shape[0]
    per_w = T // NW
    wid = _sc_wid()

    def chunk(i, _):
        base = wid * per_w + i * COMBINE_CT
        pltpu.sync_copy(pos2.at[0, pl.ds(base, COMBINE_CT)], idxe_v)
        pltpu.sync_copy(pos2.at[1, pl.ds(base, COMBINE_CT)], idxo_v)
        pltpu.sync_copy(wtb.at[0, pl.ds(base, COMBINE_CT)], wte_v)
        pltpu.sync_copy(wtb.at[1, pl.ds(base, COMBINE_CT)], wto_v)
        c1 = pltpu.async_copy(ys.at[idxe_v], rows0_v, sem)
        c2 = pltpu.async_copy(ys.at[idxo_v], rows1_v, sem)
        c1.wait()
        c2.wait()

        def row(r, _):
            w0 = wte_v[r, :]
            w1 = wto_v[r, :]
            for k in range(D_MODEL // LANES):
                sl = pl.ds(k * LANES, LANES)
                rows0_v[r, sl] = w0 * rows0_v[r, sl] + w1 * rows1_v[r, sl]
            return 0

        lax.fori_loop(0, COMBINE_CT, row, 0)
        pltpu.sync_copy(rows0_v, out.at[pl.ds(base, COMBINE_CT)])
        return 0

    lax.fori_loop(0, per_w // COMBINE_CT, chunk, 0)


def kernel(x, Wg, w1, w3, w2):
    B, S, D = x.shape
    T = B * S
    A = T * TOP_K
    nt = A // TM
    G = nt + NUM_EXPERTS - 1

    gate_logits = jnp.einsum('bsd,de->bse', x, Wg)
    weights, selected = jax.lax.top_k(gate_logits, TOP_K)
    weights = jax.nn.softmax(weights, axis=2)

    e_flat = selected.reshape(A).astype(jnp.int32)

    CH = 128
    nch = A // CH
    onehot = (jnp.arange(NUM_EXPERTS, dtype=jnp.int32)[:, None]
              == e_flat[None, :]).astype(jnp.float32)
    ohc = onehot.reshape(NUM_EXPERTS, nch, CH)
    ii = jnp.arange(CH)
    u_incl = (ii[:, None] <= ii[None, :]).astype(jnp.float32)
    within = jnp.einsum('enc,cd->end', ohc, u_incl)
    chunk_tot = within[..., -1]
    jj = jnp.arange(nch)
    u_excl = (jj[:, None] < jj[None, :]).astype(jnp.float32)
    chunk_base = jnp.einsum('en,nm->em', chunk_tot, u_excl)
    sizes_f = chunk_tot.sum(-1)
    starts_f = jnp.concatenate([jnp.zeros((1,), jnp.float32),
                                jnp.cumsum(sizes_f)[:-1]])
    posmap = within + chunk_base[:, :, None] - 1.0 \
        + starts_f[:, None, None]
    pos = jnp.sum(posmap.reshape(NUM_EXPERTS, A) * onehot,
                  axis=0).astype(jnp.int32)
    pos2 = pos.reshape(T, TOP_K).T
    sizes = sizes_f.astype(jnp.int32)
    ends = jnp.cumsum(sizes)
    starts = ends - sizes

    t_start = starts // TM
    t_last = jnp.maximum(ends - 1, 0) // TM
    touched = jnp.where(sizes > 0, t_last - t_start + 1, 0)
    pair_end = jnp.cumsum(touched)
    pair_start = pair_end - touched

    gidx = jnp.arange(G, dtype=jnp.int32)
    e_of_g = jnp.searchsorted(pair_end, gidx, side='right').astype(jnp.int32)
    valid = e_of_g < NUM_EXPERTS
    e_cl = jnp.minimum(e_of_g, NUM_EXPERTS - 1)
    last_e = jnp.searchsorted(pair_end, pair_end[-1] - 1,
                              side='right').astype(jnp.int32)
    e_g = jnp.where(valid, e_cl, last_e)
    t_g = jnp.where(valid, t_start[e_cl] + (gidx - pair_start[e_cl]), nt - 1)
    lo_g = jnp.where(valid, jnp.maximum(starts[e_g], t_g * TM), 0)
    hi_g = jnp.where(valid, jnp.minimum(ends[e_g], (t_g + 1) * TM), 0)
    first_g = jnp.concatenate([
        jnp.ones((1,), jnp.int32),
        (t_g[1:] != t_g[:-1]).astype(jnp.int32),
    ])
    sched = jnp.stack([t_g, e_g, lo_g, hi_g, first_g])

    xf = x.reshape(T, D)
    wtb = jnp.broadcast_to(
        weights.reshape(T, TOP_K).T[:, :, None], (TOP_K, T, LANES))

    reorder = functools.partial(
        pl.kernel,
        out_type=jax.ShapeDtypeStruct((A, D_MODEL), jnp.float32),
        mesh=_SC_MESH,
        scratch_types=[
            pltpu.VMEM((REORDER_CT, D_MODEL), jnp.float32),
            pltpu.VMEM((REORDER_CT,), jnp.int32),
            pltpu.VMEM((REORDER_CT,), jnp.int32),
            pltpu.SemaphoreType.DMA,
        ],
    )(_reorder_body)
    x_sorted = reorder(xf, pos2)

    grid_spec = pltpu.PrefetchScalarGridSpec(
        num_scalar_prefetch=1,
        grid=(G,),
        in_specs=[
            pl.BlockSpec((TM, D_MODEL), lambda g, s: (s[0, g], 0)),
            pl.BlockSpec((1, D_MODEL, D_FF), lambda g, s: (s[1, g], 0, 0)),
            pl.BlockSpec((1, D_MODEL, D_FF), lambda g, s: (s[1, g], 0, 0)),
            pl.BlockSpec((1, D_FF, D_MODEL), lambda g, s: (s[1, g], 0, 0)),
        ],
        out_specs=pl.BlockSpec((TM, D_MODEL), lambda g, s: (s[0, g], 0)),
    )
    y_sorted = pl.pallas_call(
        _group_body,
        grid_spec=grid_spec,
        out_shape=jax.ShapeDtypeStruct((A, D_MODEL), jnp.float32),
        compiler_params=pltpu.CompilerParams(
            vmem_limit_bytes=100 * 1024 * 1024),
    )(sched, x_sorted, w1, w3, w2)

    combine = functools.partial(
        pl.kernel,
        out_type=jax.ShapeDtypeStruct((T, D_MODEL), jnp.float32),
        mesh=_SC_MESH,
        scratch_types=[
            pltpu.VMEM((COMBINE_CT,), jnp.int32),
            pltpu.VMEM((COMBINE_CT,), jnp.int32),
            pltpu.VMEM((COMBINE_CT, LANES), jnp.float32),
            pltpu.VMEM((COMBINE_CT, LANES), jnp.float32),
            pltpu.VMEM((COMBINE_CT, D_MODEL), jnp.float32),
            pltpu.VMEM((COMBINE_CT, D_MODEL), jnp.float32),
            pltpu.SemaphoreType.DMA,
        ],
    )(_combine_body)
    out = combine(y_sorted, pos2, wtb)

    return out.reshape(B, S, D)
